# baseline (speedup 1.0000x reference)
import functools

import jax
import jax.numpy as jnp
from jax import lax
from jax.experimental import pallas as pl
from jax.experimental.pallas import tpu as pltpu
from jax.experimental.pallas import tpu_sc as plsc

NUM_CORES = 2
NUM_SUBCORES = 16
LANES = 16
NW = NUM_CORES * NUM_SUBCORES

BATCH = 4096
FEAT = 512
NUM_CLASSES = 1000

TC_BLK = 512
B_SC = 1536
N_TC = (BATCH - B_SC) // TC_BLK

ROWS_PER_W = B_SC // NW
CHUNK = 16
NCHUNK = ROWS_PER_W // CHUNK
GROUPS = FEAT // LANES
UNROLL = 8
STEPS = GROUPS // UNROLL

_mesh = plsc.VectorSubcoreMesh(core_axis_name="c", subcore_axis_name="s")


@functools.partial(
    pl.kernel,
    out_type=jax.ShapeDtypeStruct((NW, LANES), jnp.float32),
    mesh=_mesh,
    scratch_types=[
        pltpu.VMEM((ROWS_PER_W,), jnp.int32),
        pltpu.VMEM((NCHUNK, CHUNK, FEAT), jnp.float32),
        pltpu.VMEM((NCHUNK, CHUNK, FEAT), jnp.float32),
        pltpu.VMEM((LANES,), jnp.float32),
        pltpu.SemaphoreType.DMA,
        pltpu.SemaphoreType.DMA,
        pltpu.SemaphoreType.DMA,
    ],
)
def _sc_partials(features_hbm, labels_hbm, centers_hbm, out_hbm,
                 idx_v, feat_v, cent_v, acc_v, sem0, sem1, sem2):
    wid = lax.axis_index("s") * NUM_CORES + lax.axis_index("c")
    base = wid * ROWS_PER_W
    sems = (sem0, sem1, sem2)

    pltpu.sync_copy(labels_hbm.at[pl.ds(base, ROWS_PER_W)], idx_v)

    def issue(c):
        g = pltpu.async_copy(
            centers_hbm.at[idx_v.at[pl.ds(c * CHUNK, CHUNK)]],
            cent_v.at[c], sems[c])
        f = pltpu.async_copy(
            features_hbm.at[pl.ds(base + c * CHUNK, CHUNK)],
            feat_v.at[c], sems[c])
        return g, f

    inflight = [issue(c) for c in range(NCHUNK)]
    accs = (jnp.zeros((LANES,), jnp.float32),) * 4
    for c in range(NCHUNK):
        for d in inflight[c]:
            d.wait()
        fv = feat_v.at[c]
        cv = cent_v.at[c]

        def row_body(r, a):
            def step(q, aa):
                a0, a1, a2, a3 = aa
                off = q * UNROLL
                ds = []
                for j in range(UNROLL):
                    col = (off + j) * LANES
                    ds.append(fv[r, pl.ds(col, LANES)] -
                              cv[r, pl.ds(col, LANES)])
                a0 = a0 + ds[0] * ds[0] + ds[4] * ds[4]
                a1 = a1 + ds[1] * ds[1] + ds[5] * ds[5]
                a2 = a2 + ds[2] * ds[2] + ds[6] * ds[6]
                a3 = a3 + ds[3] * ds[3] + ds[7] * ds[7]
                return a0, a1, a2, a3

            return lax.fori_loop(0, STEPS, step, a)

        accs = lax.fori_loop(0, CHUNK, row_body, accs)

    acc_v[...] = (accs[0] + accs[1]) + (accs[2] + accs[3])
    pltpu.sync_copy(acc_v, out_hbm.at[wid])


def _tc_loss_block(f_ref, l_ref, c_ref, o_ref, cb_ref):
    @pl.when(pl.program_id(0) == 0)
    def _():
        o_ref[...] = jnp.zeros_like(o_ref)
        cb_ref[...] = c_ref[...].astype(jnp.bfloat16)

    lbl = l_ref[...]
    ks = lax.broadcasted_iota(jnp.int32, (NUM_CLASSES, TC_BLK), 0)
    oh_t = (ks == lbl[None, :]).astype(jnp.bfloat16)
    g = lax.dot_general(oh_t, cb_ref[...],
                        dimension_numbers=(((0,), (0,)), ((), ())),
                        preferred_element_type=jnp.float32)
    d = f_ref[...] - g
    val = jnp.sum(d * d)
    cell0 = jnp.logical_and(
        lax.broadcasted_iota(jnp.int32, (NW, LANES), 0) == 0,
        lax.broadcasted_iota(jnp.int32, (NW, LANES), 1) == 0)
    o_ref[...] += jnp.where(cell0, val, 0.0)


def _tc_loss(features, labels, centers, blk0, nblk):
    return pl.pallas_call(
        _tc_loss_block,
        grid=(nblk,),
        in_specs=[
            pl.BlockSpec((TC_BLK, FEAT), lambda i: (i + blk0, 0)),
            pl.BlockSpec((TC_BLK,), lambda i: (i + blk0,)),
            pl.BlockSpec((NUM_CLASSES, FEAT), lambda i: (0, 0)),
        ],
        out_specs=pl.BlockSpec((NW, LANES), lambda i: (0, 0)),
        out_shape=jax.ShapeDtypeStruct((NW, LANES), jnp.float32),
        scratch_shapes=[pltpu.VMEM((NUM_CLASSES, FEAT), jnp.bfloat16)],
    )(features, labels, centers)


def kernel(features, labels, centers):
    partials = _sc_partials(features, labels, centers)
    tc = _tc_loss(features, labels, centers, B_SC // TC_BLK, N_TC)
    return jnp.sum(partials + tc) * (0.5 / BATCH)


# speedup vs baseline: 1.0210x; 1.0210x over previous
import functools

import jax
import jax.numpy as jnp
from jax import lax
from jax.experimental import pallas as pl
from jax.experimental.pallas import tpu as pltpu
from jax.experimental.pallas import tpu_sc as plsc

NUM_CORES = 2
NUM_SUBCORES = 16
LANES = 16
NW = NUM_CORES * NUM_SUBCORES

BATCH = 4096
FEAT = 512
NUM_CLASSES = 1000

TC_BLK = 512
B_SC = 1536
N_TC = (BATCH - B_SC) // TC_BLK

ROWS_PER_W = B_SC // NW
CHUNK = 16
NCHUNK = ROWS_PER_W // CHUNK
GROUPS = FEAT // LANES
UNROLL = 8
STEPS = GROUPS // UNROLL

_mesh = plsc.VectorSubcoreMesh(core_axis_name="c", subcore_axis_name="s")


@functools.partial(
    pl.kernel,
    out_type=jax.ShapeDtypeStruct((NW, LANES), jnp.float32),
    mesh=_mesh,
    scratch_types=[
        pltpu.VMEM((ROWS_PER_W,), jnp.int32),
        pltpu.VMEM((2, CHUNK, FEAT), jnp.float32),
        pltpu.VMEM((2, CHUNK, FEAT), jnp.float32),
        pltpu.VMEM((LANES,), jnp.float32),
        pltpu.SemaphoreType.DMA,
        pltpu.SemaphoreType.DMA,
    ],
)
def _sc_partials(features_hbm, labels_hbm, centers_hbm, out_hbm,
                 idx_v, feat_v, cent_v, acc_v, sem0, sem1):
    wid = lax.axis_index("s") * NUM_CORES + lax.axis_index("c")
    base = wid * ROWS_PER_W
    sems = (sem0, sem1)

    pltpu.sync_copy(labels_hbm.at[pl.ds(base, ROWS_PER_W)], idx_v)

    def issue(c):
        slot = c % 2
        g = pltpu.async_copy(
            centers_hbm.at[idx_v.at[pl.ds(c * CHUNK, CHUNK)]],
            cent_v.at[slot], sems[slot])
        f = pltpu.async_copy(
            features_hbm.at[pl.ds(base + c * CHUNK, CHUNK)],
            feat_v.at[slot], sems[slot])
        return g, f

    inflight = issue(0)
    accs = (jnp.zeros((LANES,), jnp.float32),) * 4
    for c in range(NCHUNK):
        nxt = issue(c + 1) if c + 1 < NCHUNK else None
        for d in inflight:
            d.wait()
        inflight = nxt
        fv = feat_v.at[c % 2]
        cv = cent_v.at[c % 2]

        def row_body(r, a):
            def step(q, aa):
                a0, a1, a2, a3 = aa
                off = q * UNROLL
                ds = []
                for j in range(UNROLL):
                    col = (off + j) * LANES
                    ds.append(fv[r, pl.ds(col, LANES)] -
                              cv[r, pl.ds(col, LANES)])
                a0 = a0 + ds[0] * ds[0] + ds[4] * ds[4]
                a1 = a1 + ds[1] * ds[1] + ds[5] * ds[5]
                a2 = a2 + ds[2] * ds[2] + ds[6] * ds[6]
                a3 = a3 + ds[3] * ds[3] + ds[7] * ds[7]
                return a0, a1, a2, a3

            return lax.fori_loop(0, STEPS, step, a)

        accs = lax.fori_loop(0, CHUNK, row_body, accs)

    acc_v[...] = (accs[0] + accs[1]) + (accs[2] + accs[3])
    pltpu.sync_copy(acc_v, out_hbm.at[wid])


def _tc_loss_block(f_ref, l_ref, c_ref, o_ref, cb_ref):
    @pl.when(pl.program_id(0) == 0)
    def _():
        o_ref[...] = jnp.zeros_like(o_ref)
        cb_ref[...] = c_ref[...].astype(jnp.bfloat16)

    lbl = l_ref[...]
    ks = lax.broadcasted_iota(jnp.int32, (NUM_CLASSES, TC_BLK), 0)
    oh_t = (ks == lbl[None, :]).astype(jnp.bfloat16)
    g = lax.dot_general(oh_t, cb_ref[...],
                        dimension_numbers=(((0,), (0,)), ((), ())),
                        preferred_element_type=jnp.float32)
    d = f_ref[...] - g
    val = jnp.sum(d * d)
    cell0 = jnp.logical_and(
        lax.broadcasted_iota(jnp.int32, (NW, LANES), 0) == 0,
        lax.broadcasted_iota(jnp.int32, (NW, LANES), 1) == 0)
    o_ref[...] += jnp.where(cell0, val, 0.0)


def _tc_loss(features, labels, centers, blk0, nblk):
    return pl.pallas_call(
        _tc_loss_block,
        grid=(nblk,),
        in_specs=[
            pl.BlockSpec((TC_BLK, FEAT), lambda i: (i + blk0, 0)),
            pl.BlockSpec((TC_BLK,), lambda i: (i + blk0,)),
            pl.BlockSpec((NUM_CLASSES, FEAT), lambda i: (0, 0)),
        ],
        out_specs=pl.BlockSpec((NW, LANES), lambda i: (0, 0)),
        out_shape=jax.ShapeDtypeStruct((NW, LANES), jnp.float32),
        scratch_shapes=[pltpu.VMEM((NUM_CLASSES, FEAT), jnp.bfloat16)],
    )(features, labels, centers)


def kernel(features, labels, centers):
    partials = _sc_partials(features, labels, centers)
    tc = _tc_loss(features, labels, centers, B_SC // TC_BLK, N_TC)
    return jnp.sum(partials + tc) * (0.5 / BATCH)
